# baseline (speedup 1.0000x reference)
import functools

import jax
import jax.numpy as jnp
from jax import lax
from jax.experimental import pallas as pl
from jax.experimental.pallas import tpu as pltpu
from jax.experimental.pallas import tpu_sc as plsc

D = 64
NW = 32
CHUNK = 512


@functools.partial(jax.jit, static_argnums=(2,))
def _sc_gather(table, idx_flat, n):
    bpw = n // NW
    nchunk = bpw // CHUNK
    mesh = plsc.VectorSubcoreMesh(core_axis_name="c", subcore_axis_name="s")

    @functools.partial(
        pl.kernel,
        out_type=jax.ShapeDtypeStruct((n, D), jnp.float32),
        mesh=mesh,
        scratch_types=[
            pltpu.VMEM((bpw,), jnp.int32),
            pltpu.VMEM((2, CHUNK, D), jnp.float32),
            pltpu.SemaphoreType.DMA,
            pltpu.SemaphoreType.DMA,
        ],
        compiler_params=pltpu.CompilerParams(use_tc_tiling_on_sc=False),
    )
    def k(table_hbm, idx_hbm, out_hbm, idx_v, rows_v, gsem, wsem):
        wid = lax.axis_index("s") * 2 + lax.axis_index("c")
        base = wid * bpw
        pltpu.sync_copy(idx_hbm.at[pl.ds(base, bpw)], idx_v)

        def gather(c, b, start):
            cp = pltpu.make_async_copy(
                table_hbm.at[idx_v.at[pl.ds(c * CHUNK, CHUNK)]],
                rows_v.at[b],
                gsem,
            )
            cp.start() if start else cp.wait()

        def write(c, b, start):
            cp = pltpu.make_async_copy(
                rows_v.at[b],
                out_hbm.at[pl.ds(base + c * CHUNK, CHUNK)],
                wsem,
            )
            cp.start() if start else cp.wait()

        gather(0, 0, True)

        def body(o, carry):
            for b in range(2):
                c = o * 2 + b
                nb = (b + 1) % 2
                pl.when(c >= 1)(lambda: write(c - 1, nb, False))
                pl.when(c + 1 < nchunk)(lambda: gather(c + 1, nb, True))
                gather(c, b, False)
                write(c, b, True)
            return carry

        lax.fori_loop(0, nchunk // 2, body, 0)
        write(nchunk - 1, (nchunk - 1) % 2, False)

    return k(table, idx_flat)


def kernel(char_idx, table):
    b, s, t = char_idx.shape
    n = b * s * t
    idx_flat = char_idx.reshape(-1).astype(jnp.int32)
    out = _sc_gather(table, idx_flat, n)
    return out.reshape(b, s, t, D)
